# baseline (speedup 1.0000x reference)
import functools

import jax
import jax.numpy as jnp
from jax import lax
from jax.experimental import pallas as pl
from jax.experimental.pallas import tpu as pltpu
from jax.experimental.pallas import tpu_sc as plsc

NC = 2
NS = 16
LN = 16
NW = NC * NS
CK = 128


def _mm_body(x_ref, w_ref, b_ref, o_ref, *, nr):
    rr = pl.program_id(1)
    acc = jnp.dot(x_ref[...], w_ref[0], preferred_element_type=jnp.float32)
    o_ref[...] = acc + jnp.where(rr == nr - 1, 1.0, 0.0) * b_ref[...]


def _mm(x, wstack, bias, *, bn=2000):
    n, d = x.shape
    nr = wstack.shape[0]
    nb = n // bn
    return pl.pallas_call(
        functools.partial(_mm_body, nr=nr),
        grid=(nb, nr),
        in_specs=[
            pl.BlockSpec((bn, d), lambda i, rr: (i, 0)),
            pl.BlockSpec((1, d, d), lambda i, rr: (rr, 0, 0)),
            pl.BlockSpec((1, d), lambda i, rr: (0, 0)),
        ],
        out_specs=pl.BlockSpec((bn, d), lambda i, rr: (rr * nb + i, 0)),
        out_shape=jax.ShapeDtypeStruct((nr * n, d), jnp.float32),
        compiler_params=pltpu.CompilerParams(
            dimension_semantics=("parallel", "arbitrary")),
    )(x, wstack, bias.reshape(1, d))


def _combine_body(p0_ref, pr_ref, o_ref):
    o_ref[...] = p0_ref[...] + pr_ref[...]


def _combine(part, p_all, n, d, nr, *, bn=2000):
    nb = n // bn
    off = (nr - 1) * nb
    return pl.pallas_call(
        _combine_body,
        grid=(nb,),
        in_specs=[
            pl.BlockSpec((bn, d), lambda i: (i, 0)),
            pl.BlockSpec((bn, d), lambda i: (off + i, 0)),
        ],
        out_specs=pl.BlockSpec((bn, d), lambda i: (i, 0)),
        out_shape=jax.ShapeDtypeStruct((n, d), jnp.float32),
    )(part, p_all)


def _norm_body(dst_hbm, et_hbm, norm_hbm,
               cnt_sh, cbuf, dst_v, et_v, k_v, ones_v, nbuf, sem,
               *, nr_rel, n, ept, nbpad):
    cc = lax.axis_index("c")
    s = lax.axis_index("s")
    wid = cc * NS + s
    nbins = nr_rel * n
    zs = nbpad // NS
    lo = s * zs
    z16 = jnp.zeros((LN,), jnp.float32)

    def _zb(i, _):
        cbuf[pl.ds(i * LN, LN)] = z16
        return 0
    lax.fori_loop(0, zs // LN, _zb, 0)
    pltpu.sync_copy(cbuf, cnt_sh.at[pl.ds(lo, zs)])

    def _ob(i, _):
        ones_v[pl.ds(i * LN, LN)] = jnp.ones((LN,), jnp.float32)
        return 0
    lax.fori_loop(0, CK // LN, _ob, 0)
    plsc.subcore_barrier()

    def _count_row(row):
        base = row * ept

        def _ch(t, _):
            o = base + t * CK
            pltpu.sync_copy(dst_hbm.at[pl.ds(o, CK)], dst_v)
            pltpu.sync_copy(et_hbm.at[pl.ds(o, CK)], et_v)
            for j in range(CK // LN):
                sl = pl.ds(j * LN, LN)
                k_v[sl] = dst_v[sl] * nr_rel + et_v[sl]
            pltpu.sync_copy(ones_v, cnt_sh.at[k_v], add=True)
            return 0
        lax.fori_loop(0, ept // CK, _ch, 0)

    _count_row(2 * s)
    _count_row(2 * s + 1)
    plsc.subcore_barrier()

    pltpu.sync_copy(cnt_sh.at[pl.ds(lo, zs)], cbuf)

    def _inv(i, _):
        sl = pl.ds(i * LN, LN)
        cv = cbuf[sl]
        bin0 = lo + i * LN + lax.iota(jnp.int32, LN)
        iv = 1.0 / jnp.maximum(cv, 1.0)
        cbuf[sl] = jnp.where(bin0 < nbins, iv, 0.0)
        return 0
    lax.fori_loop(0, zs // LN, _inv, 0)
    pltpu.sync_copy(cbuf, cnt_sh.at[pl.ds(lo, zs)])
    plsc.subcore_barrier()

    wbase = wid * ept

    def _nch(t, _):
        o = wbase + t * CK
        pltpu.sync_copy(dst_hbm.at[pl.ds(o, CK)], dst_v)
        pltpu.sync_copy(et_hbm.at[pl.ds(o, CK)], et_v)
        for j in range(CK // LN):
            sl = pl.ds(j * LN, LN)
            k_v[sl] = dst_v[sl] * nr_rel + et_v[sl]
        pltpu.async_copy(cnt_sh.at[k_v], nbuf, sem).wait()
        pltpu.sync_copy(nbuf, norm_hbm.at[pl.ds(o, CK)])
        return 0
    lax.fori_loop(0, ept // CK, _nch, 0)


def _norm_sc(dst_p, et_p, *, nr_rel, n, ept):
    n_pad = -(-n // (NS * 16)) * (NS * 16)
    nbpad = -(-(nr_rel * n_pad) // (NS * 128)) * (NS * 128)
    zs = nbpad // NS
    mesh = plsc.VectorSubcoreMesh(core_axis_name="c", subcore_axis_name="s",
                                  num_cores=NC, num_subcores=NS)
    fn = pl.kernel(
        functools.partial(_norm_body, nr_rel=nr_rel, n=n, ept=ept, nbpad=nbpad),
        out_type=jax.ShapeDtypeStruct((NW * ept,), jnp.float32),
        mesh=mesh,
        compiler_params=pltpu.CompilerParams(needs_layout_passes=False),
        scratch_types=[
            pltpu.VMEM_SHARED((nbpad,), jnp.float32),
            pltpu.VMEM((zs,), jnp.float32),
            pltpu.VMEM((CK,), jnp.int32),
            pltpu.VMEM((CK,), jnp.int32),
            pltpu.VMEM((CK,), jnp.int32),
            pltpu.VMEM((CK,), jnp.float32),
            pltpu.VMEM((CK,), jnp.float32),
            pltpu.SemaphoreType.DMA,
        ],
    )
    return fn(dst_p, et_p)


def _agg_body(p_hbm, pk_hbm, nrm_hbm, out_hbm,
              acc_sh,
              eb0, eb1, g0, g1, ds0, ds1, nm0, nm1, rw0, rw1, zrow,
              si0, si1, sg0, sg1, ss0, ss1,
              *, n, d, nch0, n_pad, wb0, wb1):
    cc = lax.axis_index("c")
    s = lax.axis_index("s")
    zr = zrow.shape[0]
    nz = n_pad // NS
    z16 = jnp.zeros((LN,), jnp.float32)
    cbase = s * nch0
    wbase = cbase * (3 * CK)
    nbase = cbase * CK

    bufs = ((eb0, g0, ds0, nm0, rw0, si0, sg0, ss0),
            (eb1, g1, ds1, nm1, rw1, si1, sg1, ss1))

    def _idx_start(t, b):
        eb, _, _, _, _, si, _, _ = bufs[b]
        pltpu.async_copy(pk_hbm.at[pl.ds(wbase + t * (3 * CK), 3 * CK)], eb, si)

    def _nrm_start(t, b):
        _, _, _, nm, _, si, _, _ = bufs[b]
        pltpu.async_copy(nrm_hbm.at[pl.ds(nbase + t * CK, CK)],
                         nm.at[pl.ds(0, CK)], si)

    def _decode(b):
        eb, g, dsv, nm, _, si, _, _ = bufs[b]
        pltpu.make_async_copy(pk_hbm.at[pl.ds(wbase, 3 * CK)], eb, si).wait()
        pltpu.make_async_copy(nrm_hbm.at[pl.ds(nbase, CK)],
                              nm.at[pl.ds(0, CK)], si).wait()
        for j in range(CK // LN):
            sl = pl.ds(j * LN, LN)
            g[sl] = eb[pl.ds(CK + j * LN, LN)] * n + eb[sl]
            dsv[sl] = eb[pl.ds(2 * CK + j * LN, LN)]

    def _gather_start(b):
        _, g, _, _, rw, _, sg, _ = bufs[b]
        pltpu.async_copy(p_hbm.at[g], rw, sg)

    def _scale_scatter(t, b, nch):
        _, g, dsv, nm, rw, _, sg, ss = bufs[b]
        pltpu.make_async_copy(p_hbm.at[g], rw, sg).wait()

        def _eb(ei, _):
            sc = nm[pl.ds(ei, LN)][0]
            for j in range(d // LN):
                sl = pl.ds(j * LN, LN)
                rw[ei, sl] = rw[ei, sl] * sc
            return 0
        lax.fori_loop(0, CK, _eb, 0, unroll=2)

        @pl.when(t + 2 < nch)
        def _():
            _nrm_start(t + 2, b)
        pltpu.async_copy(rw, acc_sh.at[dsv], ss, add=True)

    def _scatter_wait(b):
        _, _, dsv, _, rw, _, _, ss = bufs[b]
        pltpu.make_async_copy(rw, acc_sh.at[dsv], ss).wait()

    def _go(nch):
        _idx_start(0, 0)
        _idx_start(1, 1)
        _nrm_start(0, 0)
        _nrm_start(1, 1)

        def _pair(i, _):
            for b in range(2):
                t = 2 * i + b

                @pl.when(t >= 2)
                def _():
                    _scatter_wait(b)
                _decode(b)
                _gather_start(b)

                @pl.when(t + 2 < nch)
                def _():
                    _idx_start(t + 2, b)

                @pl.when(t >= 1)
                def _():
                    _scale_scatter(t - 1, 1 - b, nch)
            return 0
        lax.fori_loop(0, nch // 2, _pair, 0)
        _scale_scatter(nch - 1, 1, nch)
        _scatter_wait(0)
        _scatter_wait(1)

    @pl.when(cc == 0)
    def _sc0_all():
        def _zb(i, _):
            for j in range(d // LN):
                zrow[i, pl.ds(j * LN, LN)] = z16
            return 0
        lax.fori_loop(0, zr, _zb, 0)

        def _zc(i, _):
            pltpu.sync_copy(zrow, acc_sh.at[pl.ds(s * nz + i * zr, zr)])
            return 0
        lax.fori_loop(0, nz // zr, _zc, 0)
        plsc.subcore_barrier()
        _go(nch0)
        plsc.subcore_barrier()

        @pl.when(s < NS - 1)
        def _wb_main():
            pltpu.sync_copy(acc_sh.at[pl.ds(s * wb0, wb0)],
                            out_hbm.at[pl.ds(s * wb0, wb0)])

        @pl.when(s == NS - 1)
        def _wb_last():
            pltpu.sync_copy(acc_sh.at[pl.ds((NS - 1) * wb0, wb1)],
                            out_hbm.at[pl.ds((NS - 1) * wb0, wb1)])


def _agg_sc(p_all, packed, norm2, *, n, d, nch0):
    n_pad = -(-n // (NS * 16)) * (NS * 16)
    wb0 = -(-n // NS // 8) * 8
    wb1 = n - (NS - 1) * wb0
    mesh = plsc.VectorSubcoreMesh(core_axis_name="c", subcore_axis_name="s",
                                  num_cores=NC, num_subcores=NS)
    fn = pl.kernel(
        functools.partial(_agg_body, n=n, d=d, nch0=nch0,
                          n_pad=n_pad, wb0=wb0, wb1=wb1),
        out_type=jax.ShapeDtypeStruct((n, d), jnp.float32),
        mesh=mesh,
        compiler_params=pltpu.CompilerParams(needs_layout_passes=False),
        scratch_types=[
            pltpu.VMEM_SHARED((n_pad, d), jnp.float32),
            pltpu.VMEM((3 * CK,), jnp.int32),
            pltpu.VMEM((3 * CK,), jnp.int32),
            pltpu.VMEM((CK,), jnp.int32),
            pltpu.VMEM((CK,), jnp.int32),
            pltpu.VMEM((CK,), jnp.int32),
            pltpu.VMEM((CK,), jnp.int32),
            pltpu.VMEM((CK + LN,), jnp.float32),
            pltpu.VMEM((CK + LN,), jnp.float32),
            pltpu.VMEM((CK, d), jnp.float32),
            pltpu.VMEM((CK, d), jnp.float32),
            pltpu.VMEM((16, d), jnp.float32),
            pltpu.SemaphoreType.DMA,
            pltpu.SemaphoreType.DMA,
            pltpu.SemaphoreType.DMA,
            pltpu.SemaphoreType.DMA,
            pltpu.SemaphoreType.DMA,
            pltpu.SemaphoreType.DMA,
        ],
    )
    return fn(p_all, packed, norm2)


def kernel(x, edge_index, edge_type, W1, root1, bias1, W2, root2, bias2):
    n, d = x.shape
    e = edge_type.shape[0]
    nr_rel = W1.shape[0]
    src, dst = edge_index[0], edge_index[1]

    ept = -(-e // (NW * 2 * CK)) * (2 * CK)
    nch = ept // CK
    epad = NW * ept - e
    pad0 = jnp.zeros((epad,), jnp.int32)
    src_p = jnp.concatenate([src, pad0])
    et_p = jnp.concatenate([edge_type, pad0])
    n_pad = -(-n // (NS * 16)) * (NS * 16)
    pad_dst = n + jnp.arange(epad, dtype=jnp.int32) % jnp.int32(n_pad - n)
    dst_p = jnp.concatenate([dst, pad_dst])
    packed = jnp.stack([a.reshape(-1, CK) for a in (src_p, et_p, dst_p)],
                       axis=1).reshape(-1)
    nch0 = NW * ept // CK // NS

    norm2 = _norm_sc(dst_p, et_p, nr_rel=nr_rel, n=n, ept=ept)

    ws1 = jnp.concatenate([W1, root1[None]], axis=0)
    p1 = _mm(x, ws1, bias1)
    part1 = _agg_sc(p1, packed, norm2, n=n, d=d, nch0=nch0)
    h = _combine(part1, p1, n, d, nr_rel + 1)

    ws2 = jnp.concatenate([W2, root2[None]], axis=0)
    p2 = _mm(h, ws2, bias2)
    part2 = _agg_sc(p2, packed, norm2, n=n, d=d, nch0=nch0)
    return _combine(part2, p2, n, d, nr_rel + 1)


# speedup vs baseline: 1.2932x; 1.2932x over previous
import functools

import jax
import jax.numpy as jnp
from jax import lax
from jax.experimental import pallas as pl
from jax.experimental.pallas import tpu as pltpu
from jax.experimental.pallas import tpu_sc as plsc

NC = 2
NS = 16
LN = 16
NW = NC * NS
CK = 128


def _mm_body(x_ref, w_ref, b_ref, o_ref, *, nr):
    rr = pl.program_id(1)
    acc = jnp.dot(x_ref[...], w_ref[0], preferred_element_type=jnp.float32)
    o_ref[...] = acc + jnp.where(rr == nr - 1, 1.0, 0.0) * b_ref[...]


def _mm(x, wstack, bias, *, bn=2000):
    n, d = x.shape
    nr = wstack.shape[0]
    nb = n // bn
    return pl.pallas_call(
        functools.partial(_mm_body, nr=nr),
        grid=(nb, nr),
        in_specs=[
            pl.BlockSpec((bn, d), lambda i, rr: (i, 0)),
            pl.BlockSpec((1, d, d), lambda i, rr: (rr, 0, 0)),
            pl.BlockSpec((1, d), lambda i, rr: (0, 0)),
        ],
        out_specs=pl.BlockSpec((bn, d), lambda i, rr: (rr * nb + i, 0)),
        out_shape=jax.ShapeDtypeStruct((nr * n, d), jnp.float32),
        compiler_params=pltpu.CompilerParams(
            dimension_semantics=("parallel", "arbitrary")),
    )(x, wstack, bias.reshape(1, d))


def _combine_body(p0_ref, p1_ref, pr_ref, o_ref):
    o_ref[...] = p0_ref[...] + p1_ref[...] + pr_ref[...]


def _combine(part, p_all, n, d, nr, *, bn=2000):
    nb = n // bn
    off = (nr - 1) * nb
    return pl.pallas_call(
        _combine_body,
        grid=(nb,),
        in_specs=[
            pl.BlockSpec((bn, d), lambda i: (i, 0)),
            pl.BlockSpec((bn, d), lambda i: (nb + i, 0)),
            pl.BlockSpec((bn, d), lambda i: (off + i, 0)),
        ],
        out_specs=pl.BlockSpec((bn, d), lambda i: (i, 0)),
        out_shape=jax.ShapeDtypeStruct((n, d), jnp.float32),
    )(part, part, p_all)


def _norm_body(dst_hbm, et_hbm, norm_hbm,
               cnt_sh, cbuf, dst_v, et_v, k_v, ones_v, nbuf, sem,
               *, nr_rel, n, ept, nbpad):
    cc = lax.axis_index("c")
    s = lax.axis_index("s")
    wid = cc * NS + s
    nbins = nr_rel * n
    zs = nbpad // NS
    lo = s * zs
    z16 = jnp.zeros((LN,), jnp.float32)

    def _zb(i, _):
        cbuf[pl.ds(i * LN, LN)] = z16
        return 0
    lax.fori_loop(0, zs // LN, _zb, 0)
    pltpu.sync_copy(cbuf, cnt_sh.at[pl.ds(lo, zs)])

    def _ob(i, _):
        ones_v[pl.ds(i * LN, LN)] = jnp.ones((LN,), jnp.float32)
        return 0
    lax.fori_loop(0, CK // LN, _ob, 0)
    plsc.subcore_barrier()

    def _count_row(row):
        base = row * ept

        def _ch(t, _):
            o = base + t * CK
            pltpu.sync_copy(dst_hbm.at[pl.ds(o, CK)], dst_v)
            pltpu.sync_copy(et_hbm.at[pl.ds(o, CK)], et_v)
            for j in range(CK // LN):
                sl = pl.ds(j * LN, LN)
                k_v[sl] = dst_v[sl] * nr_rel + et_v[sl]
            pltpu.sync_copy(ones_v, cnt_sh.at[k_v], add=True)
            return 0
        lax.fori_loop(0, ept // CK, _ch, 0)

    _count_row(2 * s)
    _count_row(2 * s + 1)
    plsc.subcore_barrier()

    pltpu.sync_copy(cnt_sh.at[pl.ds(lo, zs)], cbuf)

    def _inv(i, _):
        sl = pl.ds(i * LN, LN)
        cv = cbuf[sl]
        bin0 = lo + i * LN + lax.iota(jnp.int32, LN)
        iv = 1.0 / jnp.maximum(cv, 1.0)
        cbuf[sl] = jnp.where(bin0 < nbins, iv, 0.0)
        return 0
    lax.fori_loop(0, zs // LN, _inv, 0)
    pltpu.sync_copy(cbuf, cnt_sh.at[pl.ds(lo, zs)])
    plsc.subcore_barrier()

    wbase = wid * ept

    def _nch(t, _):
        o = wbase + t * CK
        pltpu.sync_copy(dst_hbm.at[pl.ds(o, CK)], dst_v)
        pltpu.sync_copy(et_hbm.at[pl.ds(o, CK)], et_v)
        for j in range(CK // LN):
            sl = pl.ds(j * LN, LN)
            k_v[sl] = dst_v[sl] * nr_rel + et_v[sl]
        pltpu.async_copy(cnt_sh.at[k_v], nbuf, sem).wait()
        pltpu.sync_copy(nbuf, norm_hbm.at[pl.ds(o, CK)])
        return 0
    lax.fori_loop(0, ept // CK, _nch, 0)


def _norm_sc(dst_p, et_p, *, nr_rel, n, ept):
    n_pad = -(-n // (NS * 16)) * (NS * 16)
    nbpad = -(-(nr_rel * n_pad) // (NS * 128)) * (NS * 128)
    zs = nbpad // NS
    mesh = plsc.VectorSubcoreMesh(core_axis_name="c", subcore_axis_name="s",
                                  num_cores=NC, num_subcores=NS)
    fn = pl.kernel(
        functools.partial(_norm_body, nr_rel=nr_rel, n=n, ept=ept, nbpad=nbpad),
        out_type=jax.ShapeDtypeStruct((NW * ept,), jnp.float32),
        mesh=mesh,
        compiler_params=pltpu.CompilerParams(needs_layout_passes=False),
        scratch_types=[
            pltpu.VMEM_SHARED((nbpad,), jnp.float32),
            pltpu.VMEM((zs,), jnp.float32),
            pltpu.VMEM((CK,), jnp.int32),
            pltpu.VMEM((CK,), jnp.int32),
            pltpu.VMEM((CK,), jnp.int32),
            pltpu.VMEM((CK,), jnp.float32),
            pltpu.VMEM((CK,), jnp.float32),
            pltpu.SemaphoreType.DMA,
        ],
    )
    return fn(dst_p, et_p)


def _agg_body(p_hbm, pk_hbm, nrm_hbm, out_hbm,
              acc_sh,
              eb0, eb1, g0, g1, ds0, ds1, nm0, nm1, rw0, rw1, zrow,
              si0, si1, sg0, sg1, ss0, ss1,
              *, n, d, nch0, nch1, n_pad, wb0, wb1):
    cc = lax.axis_index("c")
    s = lax.axis_index("s")
    wid = cc * NS + s
    zr = zrow.shape[0]
    nz = n_pad // NS
    z16 = jnp.zeros((LN,), jnp.float32)
    cbase = jnp.where(wid < NS, wid * nch0, NS * nch0 + (wid - NS) * nch1)
    wbase = cbase * (3 * CK)
    nbase = cbase * CK

    bufs = ((eb0, g0, ds0, nm0, rw0, si0, sg0, ss0),
            (eb1, g1, ds1, nm1, rw1, si1, sg1, ss1))

    def _idx_start(t, b):
        eb, _, _, _, _, si, _, _ = bufs[b]
        pltpu.async_copy(pk_hbm.at[pl.ds(wbase + t * (3 * CK), 3 * CK)], eb, si)

    def _nrm_start(t, b):
        _, _, _, nm, _, si, _, _ = bufs[b]
        pltpu.async_copy(nrm_hbm.at[pl.ds(nbase + t * CK, CK)],
                         nm.at[pl.ds(0, CK)], si)

    def _decode(b):
        eb, g, dsv, nm, _, si, _, _ = bufs[b]
        pltpu.make_async_copy(pk_hbm.at[pl.ds(wbase, 3 * CK)], eb, si).wait()
        pltpu.make_async_copy(nrm_hbm.at[pl.ds(nbase, CK)],
                              nm.at[pl.ds(0, CK)], si).wait()
        for j in range(CK // LN):
            sl = pl.ds(j * LN, LN)
            g[sl] = eb[pl.ds(CK + j * LN, LN)] * n + eb[sl]
            dsv[sl] = eb[pl.ds(2 * CK + j * LN, LN)]

    def _gather_start(b):
        _, g, _, _, rw, _, sg, _ = bufs[b]
        pltpu.async_copy(p_hbm.at[g], rw, sg)

    def _scale_scatter(t, b, nch):
        _, g, dsv, nm, rw, _, sg, ss = bufs[b]
        pltpu.make_async_copy(p_hbm.at[g], rw, sg).wait()

        def _eb(ei, _):
            sc = nm[pl.ds(ei, LN)][0]
            for j in range(d // LN):
                sl = pl.ds(j * LN, LN)
                rw[ei, sl] = rw[ei, sl] * sc
            return 0
        lax.fori_loop(0, CK, _eb, 0, unroll=2)

        @pl.when(t + 2 < nch)
        def _():
            _nrm_start(t + 2, b)
        pltpu.async_copy(rw, acc_sh.at[dsv], ss, add=True)

    def _scatter_wait(b):
        _, _, dsv, _, rw, _, _, ss = bufs[b]
        pltpu.make_async_copy(rw, acc_sh.at[dsv], ss).wait()

    def _go(nch):
        _idx_start(0, 0)
        _idx_start(1, 1)
        _nrm_start(0, 0)
        _nrm_start(1, 1)

        def _pair(i, _):
            for b in range(2):
                t = 2 * i + b

                @pl.when(t >= 2)
                def _():
                    _scatter_wait(b)
                _decode(b)
                _gather_start(b)

                @pl.when(t + 2 < nch)
                def _():
                    _idx_start(t + 2, b)

                @pl.when(t >= 1)
                def _():
                    _scale_scatter(t - 1, 1 - b, nch)
            return 0
        lax.fori_loop(0, nch // 2, _pair, 0)
        _scale_scatter(nch - 1, 1, nch)
        _scatter_wait(0)
        _scatter_wait(1)

    def _zb(i, _):
        for j in range(d // LN):
            zrow[i, pl.ds(j * LN, LN)] = z16
        return 0
    lax.fori_loop(0, zr, _zb, 0)

    def _zc(i, _):
        pltpu.sync_copy(zrow, acc_sh.at[pl.ds(s * nz + i * zr, zr)])
        return 0
    lax.fori_loop(0, nz // zr, _zc, 0)
    plsc.subcore_barrier()

    @pl.when(cc == 0)
    def _go0():
        _go(nch0)

    @pl.when(cc == 1)
    def _go1():
        _go(nch1)
    plsc.subcore_barrier()

    @pl.when(s < NS - 1)
    def _wb_main():
        pltpu.sync_copy(acc_sh.at[pl.ds(s * wb0, wb0)],
                        out_hbm.at[pl.ds(cc * n + s * wb0, wb0)])

    @pl.when(s == NS - 1)
    def _wb_last():
        pltpu.sync_copy(acc_sh.at[pl.ds((NS - 1) * wb0, wb1)],
                        out_hbm.at[pl.ds(cc * n + (NS - 1) * wb0, wb1)])


def _agg_sc(p_all, packed, norm2, *, n, d, nch0, nch1):
    n_pad = -(-n // (NS * 16)) * (NS * 16)
    wb0 = -(-n // NS // 8) * 8
    wb1 = n - (NS - 1) * wb0
    mesh = plsc.VectorSubcoreMesh(core_axis_name="c", subcore_axis_name="s",
                                  num_cores=NC, num_subcores=NS)
    fn = pl.kernel(
        functools.partial(_agg_body, n=n, d=d, nch0=nch0, nch1=nch1,
                          n_pad=n_pad, wb0=wb0, wb1=wb1),
        out_type=jax.ShapeDtypeStruct((NC * n, d), jnp.float32),
        mesh=mesh,
        compiler_params=pltpu.CompilerParams(needs_layout_passes=False),
        scratch_types=[
            pltpu.VMEM_SHARED((n_pad, d), jnp.float32),
            pltpu.VMEM((3 * CK,), jnp.int32),
            pltpu.VMEM((3 * CK,), jnp.int32),
            pltpu.VMEM((CK,), jnp.int32),
            pltpu.VMEM((CK,), jnp.int32),
            pltpu.VMEM((CK,), jnp.int32),
            pltpu.VMEM((CK,), jnp.int32),
            pltpu.VMEM((CK + LN,), jnp.float32),
            pltpu.VMEM((CK + LN,), jnp.float32),
            pltpu.VMEM((CK, d), jnp.float32),
            pltpu.VMEM((CK, d), jnp.float32),
            pltpu.VMEM((16, d), jnp.float32),
            pltpu.SemaphoreType.DMA,
            pltpu.SemaphoreType.DMA,
            pltpu.SemaphoreType.DMA,
            pltpu.SemaphoreType.DMA,
            pltpu.SemaphoreType.DMA,
            pltpu.SemaphoreType.DMA,
        ],
    )
    return fn(p_all, packed, norm2)


def kernel(x, edge_index, edge_type, W1, root1, bias1, W2, root2, bias2):
    n, d = x.shape
    e = edge_type.shape[0]
    nr_rel = W1.shape[0]
    src, dst = edge_index[0], edge_index[1]

    ept = -(-e // (NW * 2 * CK)) * (2 * CK)
    nch = ept // CK
    epad = NW * ept - e
    pad0 = jnp.zeros((epad,), jnp.int32)
    src_p = jnp.concatenate([src, pad0])
    et_p = jnp.concatenate([edge_type, pad0])
    n_pad = -(-n // (NS * 16)) * (NS * 16)
    pad_dst = n + jnp.arange(epad, dtype=jnp.int32) % jnp.int32(n_pad - n)
    dst_p = jnp.concatenate([dst, pad_dst])
    packed = jnp.stack([a.reshape(-1, CK) for a in (src_p, et_p, dst_p)],
                       axis=1).reshape(-1)
    cpt = NW * ept // CK // NS
    nch0 = (cpt * 17 // 20) // 2 * 2
    nch1 = cpt - nch0

    norm2 = _norm_sc(dst_p, et_p, nr_rel=nr_rel, n=n, ept=ept)

    ws1 = jnp.concatenate([W1, root1[None]], axis=0)
    p1 = _mm(x, ws1, bias1)
    part1 = _agg_sc(p1, packed, norm2, n=n, d=d, nch0=nch0, nch1=nch1)
    h = _combine(part1, p1, n, d, nr_rel + 1)

    ws2 = jnp.concatenate([W2, root2[None]], axis=0)
    p2 = _mm(h, ws2, bias2)
    part2 = _agg_sc(p2, packed, norm2, n=n, d=d, nch0=nch0, nch1=nch1)
    return _combine(part2, p2, n, d, nr_rel + 1)


# speedup vs baseline: 1.4319x; 1.1072x over previous
import functools

import jax
import jax.numpy as jnp
from jax import lax
from jax.experimental import pallas as pl
from jax.experimental.pallas import tpu as pltpu
from jax.experimental.pallas import tpu_sc as plsc

NC = 2
NS = 16
LN = 16
NW = NC * NS
CK = 128


def _mm_body(x_ref, w_ref, b_ref, o_ref, *, nr):
    rr = pl.program_id(1)
    acc = jnp.dot(x_ref[...], w_ref[0], preferred_element_type=jnp.float32)
    o_ref[...] = acc + jnp.where(rr == nr - 1, 1.0, 0.0) * b_ref[...]


def _mm(x, wstack, bias, *, bn=2000):
    n, d = x.shape
    nr = wstack.shape[0]
    nb = n // bn
    return pl.pallas_call(
        functools.partial(_mm_body, nr=nr),
        grid=(nb, nr),
        in_specs=[
            pl.BlockSpec((bn, d), lambda i, rr: (i, 0)),
            pl.BlockSpec((1, d, d), lambda i, rr: (rr, 0, 0)),
            pl.BlockSpec((1, d), lambda i, rr: (0, 0)),
        ],
        out_specs=pl.BlockSpec((bn, d), lambda i, rr: (rr * nb + i, 0)),
        out_shape=jax.ShapeDtypeStruct((nr * n, d), jnp.float32),
        compiler_params=pltpu.CompilerParams(
            dimension_semantics=("parallel", "arbitrary")),
    )(x, wstack, bias.reshape(1, d))


def _combine_body(p0_ref, p1_ref, pr_ref, o_ref):
    o_ref[...] = p0_ref[...] + p1_ref[...] + pr_ref[...]


def _combine(part, p_all, n, d, nr, *, bn=2000):
    nb = n // bn
    off = (nr - 1) * nb
    return pl.pallas_call(
        _combine_body,
        grid=(nb,),
        in_specs=[
            pl.BlockSpec((bn, d), lambda i: (i, 0)),
            pl.BlockSpec((bn, d), lambda i: (nb + i, 0)),
            pl.BlockSpec((bn, d), lambda i: (off + i, 0)),
        ],
        out_specs=pl.BlockSpec((bn, d), lambda i: (i, 0)),
        out_shape=jax.ShapeDtypeStruct((n, d), jnp.float32),
    )(part, part, p_all)


def _norm_body(pk_hbm, norm_hbm,
               cnt_sh, cbuf, eb0, eb1, k0, k1, nb0, nb1, ones_v,
               si0, si1, sg0, sg1, sw0, sw1,
               *, nr_rel, n, ept, nbpad):
    cc = lax.axis_index("c")
    s = lax.axis_index("s")
    wid = cc * NS + s
    nbins = nr_rel * n
    zs = nbpad // NS
    lo = s * zs
    nchu = ept // CK
    z16 = jnp.zeros((LN,), jnp.float32)
    bufs = ((eb0, k0, nb0, si0, sg0, sw0), (eb1, k1, nb1, si1, sg1, sw1))

    def _idx_start(c, b):
        eb, _, _, si, _, _ = bufs[b]
        pltpu.async_copy(pk_hbm.at[pl.ds(c * (3 * CK), 3 * CK)], eb, si)

    def _decode(b):
        eb, kv, _, si, _, _ = bufs[b]
        pltpu.make_async_copy(pk_hbm.at[pl.ds(0, 3 * CK)], eb, si).wait()
        for j in range(CK // LN):
            sl = pl.ds(j * LN, LN)
            kv[sl] = eb[pl.ds(2 * CK + j * LN, LN)] * nr_rel + eb[pl.ds(CK + j * LN, LN)]

    def _zb(i, _):
        cbuf[pl.ds(i * LN, LN)] = z16
        return 0
    lax.fori_loop(0, zs // LN, _zb, 0)
    pltpu.sync_copy(cbuf, cnt_sh.at[pl.ds(lo, zs)])

    def _ob(i, _):
        ones_v[pl.ds(i * LN, LN)] = jnp.ones((LN,), jnp.float32)
        return 0
    lax.fori_loop(0, CK // LN, _ob, 0)
    cb_cnt = s * (2 * nchu)
    _idx_start(cb_cnt, 0)
    _idx_start(cb_cnt + 1, 1)
    plsc.subcore_barrier()

    def _cnt(t, _):
        for b in range(2):
            tt = 2 * t + b

            @pl.when(tt >= 2)
            def _():
                _, kv, _, _, _, sw = bufs[b]
                pltpu.make_async_copy(ones_v, cnt_sh.at[kv], sw).wait()
            _decode(b)

            @pl.when(tt + 2 < 2 * nchu)
            def _():
                _idx_start(cb_cnt + tt + 2, b)
            _, kv, _, _, _, sw = bufs[b]
            pltpu.async_copy(ones_v, cnt_sh.at[kv], sw, add=True)
        return 0
    lax.fori_loop(0, nchu, _cnt, 0)
    for b in range(2):
        _, kv, _, _, _, sw = bufs[b]
        pltpu.make_async_copy(ones_v, cnt_sh.at[kv], sw).wait()
    plsc.subcore_barrier()

    pltpu.sync_copy(cnt_sh.at[pl.ds(lo, zs)], cbuf)

    def _inv(i, _):
        sl = pl.ds(i * LN, LN)
        cv = cbuf[sl]
        bin0 = lo + i * LN + lax.iota(jnp.int32, LN)
        iv = 1.0 / jnp.maximum(cv, 1.0)
        cbuf[sl] = jnp.where(bin0 < nbins, iv, 0.0)
        return 0
    lax.fori_loop(0, zs // LN, _inv, 0)
    pltpu.sync_copy(cbuf, cnt_sh.at[pl.ds(lo, zs)])
    cb_nrm = wid * nchu
    _idx_start(cb_nrm, 0)
    _idx_start(cb_nrm + 1, 1)
    plsc.subcore_barrier()

    def _gwait(b):
        _, kv, nb, _, sg, _ = bufs[b]
        pltpu.make_async_copy(cnt_sh.at[kv], nb, sg).wait()

    def _nch(t, _):
        for b in range(2):
            tt = 2 * t + b
            _, kv, nb, _, sg, sw = bufs[b]

            @pl.when(tt >= 2)
            def _():
                pltpu.make_async_copy(nb, norm_hbm.at[pl.ds(0, CK)], sw).wait()
            _decode(b)
            pltpu.async_copy(cnt_sh.at[kv], nb, sg)

            @pl.when(tt + 2 < nchu)
            def _():
                _idx_start(cb_nrm + tt + 2, b)

            @pl.when(tt >= 1)
            def _():
                _, kvp, nbp, _, sgp, swp = bufs[1 - b]
                pltpu.make_async_copy(cnt_sh.at[kvp], nbp, sgp).wait()
                pltpu.async_copy(
                    nbp, norm_hbm.at[pl.ds((cb_nrm + tt - 1) * CK, CK)], swp)
        return 0
    lax.fori_loop(0, nchu // 2, _nch, 0)
    _gwait(1)
    pltpu.async_copy(nb1, norm_hbm.at[pl.ds((cb_nrm + nchu - 1) * CK, CK)], sw1)
    pltpu.make_async_copy(nb0, norm_hbm.at[pl.ds(0, CK)], sw0).wait()
    pltpu.make_async_copy(nb1, norm_hbm.at[pl.ds(0, CK)], sw1).wait()


def _norm_sc(packed, *, nr_rel, n, ept):
    n_pad = -(-n // (NS * 16)) * (NS * 16)
    nbpad = -(-(nr_rel * n_pad) // (NS * 128)) * (NS * 128)
    zs = nbpad // NS
    mesh = plsc.VectorSubcoreMesh(core_axis_name="c", subcore_axis_name="s",
                                  num_cores=NC, num_subcores=NS)
    fn = pl.kernel(
        functools.partial(_norm_body, nr_rel=nr_rel, n=n, ept=ept, nbpad=nbpad),
        out_type=jax.ShapeDtypeStruct((NW * ept,), jnp.float32),
        mesh=mesh,
        compiler_params=pltpu.CompilerParams(needs_layout_passes=False),
        scratch_types=[
            pltpu.VMEM_SHARED((nbpad,), jnp.float32),
            pltpu.VMEM((zs,), jnp.float32),
            pltpu.VMEM((3 * CK,), jnp.int32),
            pltpu.VMEM((3 * CK,), jnp.int32),
            pltpu.VMEM((CK,), jnp.int32),
            pltpu.VMEM((CK,), jnp.int32),
            pltpu.VMEM((CK,), jnp.float32),
            pltpu.VMEM((CK,), jnp.float32),
            pltpu.VMEM((CK,), jnp.float32),
            pltpu.SemaphoreType.DMA,
            pltpu.SemaphoreType.DMA,
            pltpu.SemaphoreType.DMA,
            pltpu.SemaphoreType.DMA,
            pltpu.SemaphoreType.DMA,
            pltpu.SemaphoreType.DMA,
        ],
    )
    return fn(packed)


def _agg_body(p_hbm, pk_hbm, nrm_hbm, out_hbm,
              acc_sh,
              eb0, eb1, g0, g1, ds0, ds1, nm0, nm1, rw0, rw1, zrow,
              si0, si1, sg0, sg1, ss0, ss1,
              *, n, d, nch0, nch1, n_pad, wb0, wb1):
    cc = lax.axis_index("c")
    s = lax.axis_index("s")
    wid = cc * NS + s
    zr = zrow.shape[0]
    nz = n_pad // NS
    z16 = jnp.zeros((LN,), jnp.float32)
    cbase = jnp.where(wid < NS, wid * nch0, NS * nch0 + (wid - NS) * nch1)
    wbase = cbase * (3 * CK)
    nbase = cbase * CK

    bufs = ((eb0, g0, ds0, nm0, rw0, si0, sg0, ss0),
            (eb1, g1, ds1, nm1, rw1, si1, sg1, ss1))

    def _idx_start(t, b):
        eb, _, _, _, _, si, _, _ = bufs[b]
        pltpu.async_copy(pk_hbm.at[pl.ds(wbase + t * (3 * CK), 3 * CK)], eb, si)

    def _nrm_start(t, b):
        _, _, _, nm, _, si, _, _ = bufs[b]
        pltpu.async_copy(nrm_hbm.at[pl.ds(nbase + t * CK, CK)],
                         nm.at[pl.ds(0, CK)], si)

    def _decode(b):
        eb, g, dsv, nm, _, si, _, _ = bufs[b]
        pltpu.make_async_copy(pk_hbm.at[pl.ds(wbase, 3 * CK)], eb, si).wait()
        pltpu.make_async_copy(nrm_hbm.at[pl.ds(nbase, CK)],
                              nm.at[pl.ds(0, CK)], si).wait()
        for j in range(CK // LN):
            sl = pl.ds(j * LN, LN)
            g[sl] = eb[pl.ds(CK + j * LN, LN)] * n + eb[sl]
            dsv[sl] = eb[pl.ds(2 * CK + j * LN, LN)]

    def _gather_start(b):
        _, g, _, _, rw, _, sg, _ = bufs[b]
        pltpu.async_copy(p_hbm.at[g], rw, sg)

    def _scale_scatter(t, b, nch):
        _, g, dsv, nm, rw, _, sg, ss = bufs[b]
        pltpu.make_async_copy(p_hbm.at[g], rw, sg).wait()

        def _eb(ei, _):
            sc = nm[pl.ds(ei, LN)][0]
            for j in range(d // LN):
                sl = pl.ds(j * LN, LN)
                rw[ei, sl] = rw[ei, sl] * sc
            return 0
        lax.fori_loop(0, CK, _eb, 0, unroll=2)

        @pl.when(t + 2 < nch)
        def _():
            _nrm_start(t + 2, b)
        pltpu.async_copy(rw, acc_sh.at[dsv], ss, add=True)

    def _scatter_wait(b):
        _, _, dsv, _, rw, _, _, ss = bufs[b]
        pltpu.make_async_copy(rw, acc_sh.at[dsv], ss).wait()

    def _go(nch):
        _idx_start(0, 0)
        _idx_start(1, 1)
        _nrm_start(0, 0)
        _nrm_start(1, 1)

        def _pair(i, _):
            for b in range(2):
                t = 2 * i + b

                @pl.when(t >= 2)
                def _():
                    _scatter_wait(b)
                _decode(b)
                _gather_start(b)

                @pl.when(t + 2 < nch)
                def _():
                    _idx_start(t + 2, b)

                @pl.when(t >= 1)
                def _():
                    _scale_scatter(t - 1, 1 - b, nch)
            return 0
        lax.fori_loop(0, nch // 2, _pair, 0)
        _scale_scatter(nch - 1, 1, nch)
        _scatter_wait(0)
        _scatter_wait(1)

    def _zb(i, _):
        for j in range(d // LN):
            zrow[i, pl.ds(j * LN, LN)] = z16
        return 0
    lax.fori_loop(0, zr, _zb, 0)

    def _zc(i, _):
        pltpu.sync_copy(zrow, acc_sh.at[pl.ds(s * nz + i * zr, zr)])
        return 0
    lax.fori_loop(0, nz // zr, _zc, 0)
    plsc.subcore_barrier()

    @pl.when(cc == 0)
    def _go0():
        _go(nch0)

    @pl.when(cc == 1)
    def _go1():
        _go(nch1)
    plsc.subcore_barrier()

    @pl.when(s < NS - 1)
    def _wb_main():
        pltpu.sync_copy(acc_sh.at[pl.ds(s * wb0, wb0)],
                        out_hbm.at[pl.ds(cc * n + s * wb0, wb0)])

    @pl.when(s == NS - 1)
    def _wb_last():
        pltpu.sync_copy(acc_sh.at[pl.ds((NS - 1) * wb0, wb1)],
                        out_hbm.at[pl.ds(cc * n + (NS - 1) * wb0, wb1)])


def _agg_sc(p_all, packed, norm2, *, n, d, nch0, nch1):
    n_pad = -(-n // (NS * 16)) * (NS * 16)
    wb0 = -(-n // NS // 8) * 8
    wb1 = n - (NS - 1) * wb0
    mesh = plsc.VectorSubcoreMesh(core_axis_name="c", subcore_axis_name="s",
                                  num_cores=NC, num_subcores=NS)
    fn = pl.kernel(
        functools.partial(_agg_body, n=n, d=d, nch0=nch0, nch1=nch1,
                          n_pad=n_pad, wb0=wb0, wb1=wb1),
        out_type=jax.ShapeDtypeStruct((NC * n, d), jnp.float32),
        mesh=mesh,
        compiler_params=pltpu.CompilerParams(needs_layout_passes=False),
        scratch_types=[
            pltpu.VMEM_SHARED((n_pad, d), jnp.float32),
            pltpu.VMEM((3 * CK,), jnp.int32),
            pltpu.VMEM((3 * CK,), jnp.int32),
            pltpu.VMEM((CK,), jnp.int32),
            pltpu.VMEM((CK,), jnp.int32),
            pltpu.VMEM((CK,), jnp.int32),
            pltpu.VMEM((CK,), jnp.int32),
            pltpu.VMEM((CK + LN,), jnp.float32),
            pltpu.VMEM((CK + LN,), jnp.float32),
            pltpu.VMEM((CK, d), jnp.float32),
            pltpu.VMEM((CK, d), jnp.float32),
            pltpu.VMEM((16, d), jnp.float32),
            pltpu.SemaphoreType.DMA,
            pltpu.SemaphoreType.DMA,
            pltpu.SemaphoreType.DMA,
            pltpu.SemaphoreType.DMA,
            pltpu.SemaphoreType.DMA,
            pltpu.SemaphoreType.DMA,
        ],
    )
    return fn(p_all, packed, norm2)


def kernel(x, edge_index, edge_type, W1, root1, bias1, W2, root2, bias2):
    n, d = x.shape
    e = edge_type.shape[0]
    nr_rel = W1.shape[0]
    src, dst = edge_index[0], edge_index[1]

    ept = -(-e // (NW * 2 * CK)) * (2 * CK)
    nch = ept // CK
    epad = NW * ept - e
    pad0 = jnp.zeros((epad,), jnp.int32)
    src_p = jnp.concatenate([src, pad0])
    et_p = jnp.concatenate([edge_type, pad0])
    n_pad = -(-n // (NS * 16)) * (NS * 16)
    pad_dst = n + jnp.arange(epad, dtype=jnp.int32) % jnp.int32(n_pad - n)
    dst_p = jnp.concatenate([dst, pad_dst])
    packed = jnp.stack([a.reshape(-1, CK) for a in (src_p, et_p, dst_p)],
                       axis=1).reshape(-1)
    cpt = NW * ept // CK // NS
    nch0 = (cpt * 17 // 20) // 2 * 2
    nch1 = cpt - nch0

    norm2 = _norm_sc(packed, nr_rel=nr_rel, n=n, ept=ept)

    ws1 = jnp.concatenate([W1, root1[None]], axis=0)
    p1 = _mm(x, ws1, bias1)
    part1 = _agg_sc(p1, packed, norm2, n=n, d=d, nch0=nch0, nch1=nch1)
    h = _combine(part1, p1, n, d, nr_rel + 1)

    ws2 = jnp.concatenate([W2, root2[None]], axis=0)
    p2 = _mm(h, ws2, bias2)
    part2 = _agg_sc(p2, packed, norm2, n=n, d=d, nch0=nch0, nch1=nch1)
    return _combine(part2, p2, n, d, nr_rel + 1)
